# baseline (speedup 1.0000x reference)
import functools

import jax
import jax.numpy as jnp
from jax import lax
from jax.experimental import pallas as pl
from jax.experimental.pallas import tpu as pltpu
from jax.experimental.pallas import tpu_sc as plsc

_info = plsc.get_sparse_core_info()
_NC, _NS = _info.num_cores, _info.num_subcores
_NW = _NC * _NS

_BATCH = 16384
_DIM = 32
_B_PER_W = _BATCH // _NW
_G = 8
_NBANK = 2
_NGROUP = _B_PER_W // _G

_mesh = plsc.VectorSubcoreMesh(core_axis_name="c", subcore_axis_name="s")


@functools.partial(
    pl.kernel,
    mesh=_mesh,
    out_type=jax.ShapeDtypeStruct((_DIM, _BATCH), jnp.float32),
    scratch_types=[
        pltpu.VMEM((_B_PER_W + 16,), jnp.int32),
        pltpu.VMEM((_NBANK * _G, _DIM, 128), jnp.float32),
        pltpu.VMEM((_DIM, _B_PER_W), jnp.float32),
        pltpu.SemaphoreType.DMA((_NBANK,)),
    ],
    compiler_params=pltpu.CompilerParams(needs_layout_passes=False),
)
def _embed_gather(idx_hbm, table_hbm, out_hbm, idx_v, ring, out_slab, sems):
    wid = lax.axis_index("s") * _NC + lax.axis_index("c")
    base = wid * _B_PER_W
    pltpu.sync_copy(idx_hbm.at[pl.ds(base, _B_PER_W)], idx_v.at[pl.ds(0, _B_PER_W)])

    rows_lo = lax.iota(jnp.int32, 16)
    rows_hi = rows_lo + 16
    zeros16 = jnp.zeros((16,), jnp.int32)

    def fire(xi, bank, j):
        col = pl.multiple_of((xi >> 7) * 128, 128)
        pltpu.async_copy(
            table_hbm.at[:, pl.ds(col, 128)],
            ring.at[bank * _G + j],
            sems.at[bank],
        )

    for b in range(_NBANK):
        ivp = idx_v[pl.ds(b * _G, 16)]
        for j in range(_G):
            fire(ivp[j], b, j)

    def body(g, carry):
        bank = lax.rem(g, _NBANK)
        iv = idx_v[pl.ds(g * _G, 16)]
        g_next = jnp.minimum(g + _NBANK, _NGROUP - 1)
        iv_next = idx_v[pl.ds(g_next * _G, 16)]
        for j in range(_G):
            pltpu.make_async_copy(
                table_hbm.at[:, pl.ds(0, 128)],
                ring.at[bank * _G + j],
                sems.at[bank],
            ).wait()
        for j in range(_G):
            lane = zeros16 + (iv[j] & 127)
            rvec = zeros16 + (g * _G + j)
            lo = plsc.load_gather(ring.at[bank * _G + j], [rows_lo, lane])
            plsc.store_scatter(out_slab, [rows_lo, rvec], lo)
            hi = plsc.load_gather(ring.at[bank * _G + j], [rows_hi, lane])
            plsc.store_scatter(out_slab, [rows_hi, rvec], hi)

        @pl.when(g + _NBANK < _NGROUP)
        def _():
            for j in range(_G):
                fire(iv_next[j], bank, j)

        return carry

    lax.fori_loop(0, _NGROUP, body, 0)
    pltpu.sync_copy(out_slab, out_hbm.at[:, pl.ds(base, _B_PER_W)])


def kernel(x, table):
    out_t = _embed_gather(x.astype(jnp.int32), table.T)
    return out_t.T


# speedup vs baseline: 1.0725x; 1.0725x over previous
import functools

import jax
import jax.numpy as jnp
from jax import lax
from jax.experimental import pallas as pl
from jax.experimental.pallas import tpu as pltpu
from jax.experimental.pallas import tpu_sc as plsc

_info = plsc.get_sparse_core_info()
_NC, _NS = _info.num_cores, _info.num_subcores
_NW = _NC * _NS

_BATCH = 16384
_DIM = 32
_B_PER_W = _BATCH // _NW
_G = 16
_DEPTH = 24
_NGROUP = _B_PER_W // _G

_mesh = plsc.VectorSubcoreMesh(core_axis_name="c", subcore_axis_name="s")


@functools.partial(
    pl.kernel,
    mesh=_mesh,
    out_type=jax.ShapeDtypeStruct((_DIM, _BATCH), jnp.float32),
    scratch_types=[
        pltpu.VMEM((_B_PER_W + _DEPTH + _G,), jnp.int32),
        pltpu.VMEM((_DEPTH, _DIM, 128), jnp.float32),
        pltpu.VMEM((_DIM, _B_PER_W), jnp.float32),
        pltpu.SemaphoreType.DMA((_DEPTH,)),
    ],
    compiler_params=pltpu.CompilerParams(needs_layout_passes=False),
)
def _embed_gather(idx_hbm, table_hbm, out_hbm, idx_v, ring, out_slab, sems):
    wid = lax.axis_index("s") * _NC + lax.axis_index("c")
    base = wid * _B_PER_W
    pltpu.sync_copy(idx_hbm.at[pl.ds(base, _B_PER_W)], idx_v.at[pl.ds(0, _B_PER_W)])

    rows_lo = lax.iota(jnp.int32, 16)
    rows_hi = rows_lo + 16
    zeros16 = jnp.zeros((16,), jnp.int32)

    def fire(xi, slot):
        col = pl.multiple_of((xi >> 7) * 128, 128)
        pltpu.async_copy(
            table_hbm.at[:, pl.ds(col, 128)],
            ring.at[slot],
            sems.at[slot],
        )

    for jj in range(0, _DEPTH, _G):
        ivp = idx_v[pl.ds(jj, _G)]
        for j in range(min(_G, _DEPTH - jj)):
            fire(ivp[j], jj + j)

    def body(g, carry):
        iv = idx_v[pl.ds(g * _G, _G)]
        iv_ahead = idx_v[pl.ds(g * _G + _DEPTH, _G)]
        for j in range(_G):
            r = g * _G + j
            slot = lax.rem(r, _DEPTH)
            pltpu.make_async_copy(
                table_hbm.at[:, pl.ds(0, 128)], ring.at[slot], sems.at[slot]
            ).wait()
            lane = zeros16 + (iv[j] & 127)
            rvec = zeros16 + r
            lo = plsc.load_gather(ring.at[slot], [rows_lo, lane])
            plsc.store_scatter(out_slab, [rows_lo, rvec], lo)
            hi = plsc.load_gather(ring.at[slot], [rows_hi, lane])
            plsc.store_scatter(out_slab, [rows_hi, rvec], hi)

            @pl.when(r + _DEPTH < _B_PER_W)
            def _():
                fire(iv_ahead[j], slot)

        return carry

    lax.fori_loop(0, _NGROUP, body, 0)
    pltpu.sync_copy(out_slab, out_hbm.at[:, pl.ds(base, _B_PER_W)])


def kernel(x, table):
    out_t = _embed_gather(x.astype(jnp.int32), table.T)
    return out_t.T


# speedup vs baseline: 1.0830x; 1.0098x over previous
import functools

import jax
import jax.numpy as jnp
from jax import lax
from jax.experimental import pallas as pl
from jax.experimental.pallas import tpu as pltpu
from jax.experimental.pallas import tpu_sc as plsc

_info = plsc.get_sparse_core_info()
_NC, _NS = _info.num_cores, _info.num_subcores
_NW = _NC * _NS

_BATCH = 16384
_DIM = 32
_B_PER_W = _BATCH // _NW
_G = 16
_DEPTH = 24
_NGROUP = _B_PER_W // _G

_mesh = plsc.VectorSubcoreMesh(core_axis_name="c", subcore_axis_name="s")


@functools.partial(
    pl.kernel,
    mesh=_mesh,
    out_type=jax.ShapeDtypeStruct((_DIM, _BATCH), jnp.float32),
    scratch_types=[
        pltpu.VMEM((_B_PER_W + _DEPTH + _G,), jnp.int32),
        pltpu.VMEM((_DEPTH, _DIM, 128), jnp.float32),
        pltpu.VMEM((_DIM, _B_PER_W), jnp.float32),
        pltpu.SemaphoreType.DMA((_DEPTH,)),
    ],
    compiler_params=pltpu.CompilerParams(needs_layout_passes=False),
)
def _embed_gather(idx_hbm, table_hbm, out_hbm, idx_v, ring, out_slab, sems):
    wid = lax.axis_index("s") * _NC + lax.axis_index("c")
    base = wid * _B_PER_W
    pltpu.sync_copy(idx_hbm.at[pl.ds(base, _B_PER_W)], idx_v.at[pl.ds(0, _B_PER_W)])

    rows_lo = lax.iota(jnp.int32, 16)
    rows_hi = rows_lo + 16
    zeros16 = jnp.zeros((16,), jnp.int32)

    def fire(xi, slot):
        col = pl.multiple_of((xi >> 7) * 128, 128)
        for t in range(4):
            pltpu.async_copy(
                table_hbm.at[pl.ds(t * 8, 8), pl.ds(col, 128)],
                ring.at[slot].at[pl.ds(t * 8, 8), :],
                sems.at[slot],
            )

    for jj in range(0, _DEPTH, _G):
        ivp = idx_v[pl.ds(jj, _G)]
        for j in range(min(_G, _DEPTH - jj)):
            fire(ivp[j], jj + j)

    def body(g, carry):
        iv = idx_v[pl.ds(g * _G, _G)]
        iv_ahead = idx_v[pl.ds(g * _G + _DEPTH, _G)]
        for j in range(_G):
            r = g * _G + j
            slot = lax.rem(r, _DEPTH)
            pltpu.make_async_copy(
                table_hbm.at[:, pl.ds(0, 128)], ring.at[slot], sems.at[slot]
            ).wait()
            lane = zeros16 + (iv[j] & 127)
            rvec = zeros16 + r
            lo = plsc.load_gather(ring.at[slot], [rows_lo, lane])
            plsc.store_scatter(out_slab, [rows_lo, rvec], lo)
            hi = plsc.load_gather(ring.at[slot], [rows_hi, lane])
            plsc.store_scatter(out_slab, [rows_hi, rvec], hi)

            @pl.when(r + _DEPTH < _B_PER_W)
            def _():
                fire(iv_ahead[j], slot)

        return carry

    lax.fori_loop(0, _NGROUP, body, 0)
    pltpu.sync_copy(out_slab, out_hbm.at[:, pl.ds(base, _B_PER_W)])


def kernel(x, table):
    out_t = _embed_gather(x.astype(jnp.int32), table.T)
    return out_t.T
